# baseline (speedup 1.0000x reference)
import jax
import jax.numpy as jnp
from jax.experimental import pallas as pl
from jax.experimental.pallas import tpu as pltpu


def _round_up(n: int, m: int) -> int:
    return ((n + m - 1) // m) * m


def _make_body(S: int, D_in: int, H1: int, H2: int, nsteps: int):
    M = 2 * S

    def _body(x_hbm, m_ref, w_ref, b_ref, o_ref, xbuf, sem):
        def dma_in(slot, step):
            return pltpu.make_async_copy(
                x_hbm.at[pl.ds(step * M, M), :], xbuf.at[slot], sem.at[slot])

        dma_in(0, 0).start()

        def step_fn(s, _):
            cur = jax.lax.rem(s, 2)
            nxt = jax.lax.rem(s + 1, 2)

            @pl.when(s + 1 < nsteps)
            def _():
                dma_in(nxt, s + 1).start()

            dma_in(cur, 0).wait()

            xb = xbuf[cur].astype(jnp.bfloat16)
            z1 = jnp.dot(xb, w_ref[0, :D_in, :H1],
                         preferred_element_type=jnp.float32)
            h1 = jnp.tanh((z1 + b_ref[0, :, :H1]).astype(jnp.bfloat16))
            z2 = jnp.dot(h1, w_ref[1, :H1, :H2],
                         preferred_element_type=jnp.float32)
            h2 = jnp.tanh((z2 + b_ref[1, :, :H2]).astype(jnp.bfloat16))

            mm = m_ref[pl.ds(s, 1)].reshape(2, S)
            row = jax.lax.broadcasted_iota(jnp.int32, (8, S), 0)
            sel0 = jnp.where(row == 0, mm[0:1, :], 0.0).astype(jnp.bfloat16)
            sel1 = jnp.where(row == 1, mm[1:2, :], 0.0).astype(jnp.bfloat16)
            pooled = (
                jnp.dot(sel0, h2[:S, :], preferred_element_type=jnp.float32)
                + jnp.dot(sel1, h2[S:, :],
                          preferred_element_type=jnp.float32)
            )
            lens = jnp.maximum(jnp.sum(mm, axis=1, keepdims=True), 1.0)
            o_ref[pl.ds(s, 1)] = (pooled[:2, :] / lens).reshape(1, 2, H2)
            return ()

        jax.lax.fori_loop(0, nsteps, step_fn, (), unroll=False)

    return _body


def kernel(x, mask, w0, w1, b0, b1):
    B, S, D_in = x.shape
    H1 = w0.shape[1]
    H2 = w1.shape[1]

    Din_p, H1_p, H2_p = (_round_up(d, 128) for d in (D_in, H1, H2))

    ws = jnp.zeros((2, max(Din_p, H1_p), H1_p), jnp.bfloat16)
    ws = ws.at[0, :D_in, :H1].set(w0.astype(jnp.bfloat16))
    ws = ws.at[1, :H1, :H2].set(w1.astype(jnp.bfloat16))
    bs = jnp.zeros((2, 1, H1_p), jnp.float32)
    bs = bs.at[0, :, :H1].set(b0.reshape(1, -1).astype(jnp.float32))
    bs = bs.at[1, :, :H2].set(b1.reshape(1, -1).astype(jnp.float32))

    xp = x
    mp = mask.astype(jnp.float32)
    if S % 8 or Din_p != D_in:
        Sp = _round_up(S, 8)
        xp = jnp.zeros((B, Sp, Din_p), x.dtype).at[:, :S, :D_in].set(x)
        mp = jnp.zeros((B, Sp), jnp.float32).at[:, :S].set(mp)
        S = Sp
    if B % 2:
        xp = jnp.concatenate([xp, jnp.zeros((1, S, Din_p), xp.dtype)], 0)
        mp = jnp.concatenate([mp, jnp.zeros((1, S), jnp.float32)], 0)
        B += 1
    nsteps = B // 2

    x2 = xp.reshape(B * S, Din_p)
    mp = mp.reshape(nsteps, 2, S)

    out = pl.pallas_call(
        _make_body(S, Din_p, H1_p, H2_p, nsteps),
        out_shape=jax.ShapeDtypeStruct((nsteps, 2, H2_p), jnp.float32),
        in_specs=[
            pl.BlockSpec(memory_space=pltpu.MemorySpace.HBM),
            pl.BlockSpec(memory_space=pltpu.MemorySpace.VMEM),
            pl.BlockSpec(memory_space=pltpu.MemorySpace.VMEM),
            pl.BlockSpec(memory_space=pltpu.MemorySpace.VMEM),
        ],
        out_specs=pl.BlockSpec(memory_space=pltpu.MemorySpace.VMEM),
        scratch_shapes=[
            pltpu.VMEM((2, 2 * S, Din_p), jnp.float32),
            pltpu.SemaphoreType.DMA((2,)),
        ],
        compiler_params=pltpu.CompilerParams(
            vmem_limit_bytes=56 << 20,
        ),
    )(x2, mp, ws, bs)
    return out.reshape(B, H2_p)[:x.shape[0], :H2].astype(x.dtype)
